# baseline (speedup 1.0000x reference)
import functools

import jax
import jax.numpy as jnp
from jax.experimental import pallas as pl


def _ffn_kernel(x_ref, gw_ref, uw_ref, gb_ref, ub_ref, dw_ref, db_ref, o_ref):
    ti = pl.program_id(2)
    half = gw_ref.shape[-1] // 2
    x = x_ref[0].astype(jnp.bfloat16)

    def ffn_half(j):
        sl = pl.ds(j * half, half)
        g = jnp.dot(x, gw_ref[0, :, sl].astype(jnp.bfloat16),
                    preferred_element_type=jnp.float32) + gb_ref[0, :, sl]
        u = jnp.dot(x, uw_ref[0, :, sl].astype(jnp.bfloat16),
                    preferred_element_type=jnp.float32) + ub_ref[0, :, sl]
        h = ((g * jax.nn.sigmoid(g)) * u).astype(jnp.bfloat16)
        return jnp.dot(h, dw_ref[0, sl, :].astype(jnp.bfloat16),
                       preferred_element_type=jnp.float32)

    acc0 = ffn_half(0)

    @pl.when(ti == 0)
    def _init():
        o_ref[0] = acc0 + db_ref[0]

    @pl.when(ti != 0)
    def _accum0():
        o_ref[0] += acc0

    o_ref[0] += ffn_half(1)


@functools.partial(jax.jit, static_argnames=("tm", "ti"))
def _ffn(x, gate_up_proj, down_proj, gate_up_proj_bias, down_proj_bias,
         tm: int, ti: int):
    e, t, h = x.shape
    i = down_proj.shape[1]
    n_ti = i // ti
    gub = gate_up_proj_bias.reshape(e, 1, 2 * i)
    db = down_proj_bias.reshape(e, 1, h)

    grid = (e, t // tm, n_ti)
    return pl.pallas_call(
        _ffn_kernel,
        grid=grid,
        in_specs=[
            pl.BlockSpec((1, tm, h), lambda ei, tmi, tii: (ei, tmi, 0)),
            pl.BlockSpec((1, h, ti), lambda ei, tmi, tii: (ei, 0, tii)),
            pl.BlockSpec((1, h, ti),
                         lambda ei, tmi, tii, n=n_ti: (ei, 0, tii + n)),
            pl.BlockSpec((1, 1, ti), lambda ei, tmi, tii: (ei, 0, tii)),
            pl.BlockSpec((1, 1, ti),
                         lambda ei, tmi, tii, n=n_ti: (ei, 0, tii + n)),
            pl.BlockSpec((1, ti, h), lambda ei, tmi, tii: (ei, tii, 0)),
            pl.BlockSpec((1, 1, h), lambda ei, tmi, tii: (ei, 0, 0)),
        ],
        out_specs=pl.BlockSpec((1, tm, h), lambda ei, tmi, tii: (ei, tmi, 0)),
        out_shape=jax.ShapeDtypeStruct((e, t, h), jnp.float32),
    )(x, gate_up_proj, gate_up_proj, gub, gub, down_proj, db)


def kernel(x, gate_up_proj, down_proj, gate_up_proj_bias, down_proj_bias):
    t = x.shape[1]
    i = down_proj.shape[1]
    tm = min(t, 1024)
    ti = min(i, 1024)
    return _ffn(x, gate_up_proj, down_proj, gate_up_proj_bias, down_proj_bias,
                tm, ti)


# speedup vs baseline: 1.0437x; 1.0437x over previous
import functools

import jax
import jax.numpy as jnp
from jax.experimental import pallas as pl
from jax.experimental.pallas import tpu as pltpu


def _ffn_kernel(x_ref, gw_ref, uw_ref, gb_ref, ub_ref, dw_ref, db_ref, o_ref):
    ti = pl.program_id(2)
    half = gw_ref.shape[-1] // 2
    x = x_ref[0].astype(jnp.bfloat16)

    def ffn_half(j):
        sl = pl.ds(j * half, half)
        g = jnp.dot(x, gw_ref[0, :, sl].astype(jnp.bfloat16),
                    preferred_element_type=jnp.float32) + gb_ref[0, :, sl]
        u = jnp.dot(x, uw_ref[0, :, sl].astype(jnp.bfloat16),
                    preferred_element_type=jnp.float32) + ub_ref[0, :, sl]
        h = ((g * jax.nn.sigmoid(g)) * u).astype(jnp.bfloat16)
        return jnp.dot(h, dw_ref[0, sl, :].astype(jnp.bfloat16),
                       preferred_element_type=jnp.float32)

    acc = ffn_half(0) + ffn_half(1)

    @pl.when(ti == 0)
    def _init():
        o_ref[0] = acc + db_ref[0]

    @pl.when(ti != 0)
    def _accum():
        o_ref[0] += acc


@functools.partial(jax.jit, static_argnames=("tm", "ti"))
def _ffn(x, gate_up_proj, down_proj, gate_up_proj_bias, down_proj_bias,
         tm: int, ti: int):
    e, t, h = x.shape
    i = down_proj.shape[1]
    n_ti = i // ti
    gub = gate_up_proj_bias.reshape(e, 1, 2 * i)
    db = down_proj_bias.reshape(e, 1, h)

    grid = (e, t // tm, n_ti)
    return pl.pallas_call(
        _ffn_kernel,
        grid=grid,
        in_specs=[
            pl.BlockSpec((1, tm, h), lambda ei, tmi, tii: (ei, tmi, 0)),
            pl.BlockSpec((1, h, ti), lambda ei, tmi, tii: (ei, 0, tii)),
            pl.BlockSpec((1, h, ti),
                         lambda ei, tmi, tii, n=n_ti: (ei, 0, tii + n)),
            pl.BlockSpec((1, 1, ti), lambda ei, tmi, tii: (ei, 0, tii)),
            pl.BlockSpec((1, 1, ti),
                         lambda ei, tmi, tii, n=n_ti: (ei, 0, tii + n)),
            pl.BlockSpec((1, ti, h), lambda ei, tmi, tii: (ei, tii, 0)),
            pl.BlockSpec((1, 1, h), lambda ei, tmi, tii: (ei, 0, 0)),
        ],
        out_specs=pl.BlockSpec((1, tm, h), lambda ei, tmi, tii: (ei, tmi, 0)),
        out_shape=jax.ShapeDtypeStruct((e, t, h), jnp.float32),
        compiler_params=pltpu.CompilerParams(
            dimension_semantics=("parallel", "parallel", "arbitrary")),
    )(x, gate_up_proj, gate_up_proj, gub, gub, down_proj, db)


def kernel(x, gate_up_proj, down_proj, gate_up_proj_bias, down_proj_bias):
    t = x.shape[1]
    i = down_proj.shape[1]
    tm = min(t, 1024)
    ti = min(i, 1024)
    return _ffn(x, gate_up_proj, down_proj, gate_up_proj_bias, down_proj_bias,
                tm, ti)


# speedup vs baseline: 1.0892x; 1.0436x over previous
import functools

import jax
import jax.numpy as jnp
from jax.experimental import pallas as pl
from jax.experimental.pallas import tpu as pltpu


def _ffn_kernel(x_ref, gw_ref, uw_ref, gb_ref, ub_ref, dw_ref, db_ref, o_ref):
    ti = pl.program_id(2)
    half = gw_ref.shape[-1] // 2
    x = x_ref[0].astype(jnp.bfloat16)

    def ffn_half(j):
        sl = pl.ds(j * half, half)
        g = jnp.dot(x, gw_ref[0, :, sl].astype(jnp.bfloat16),
                    preferred_element_type=jnp.float32) + gb_ref[0, :, sl]
        u = jnp.dot(x, uw_ref[0, :, sl].astype(jnp.bfloat16),
                    preferred_element_type=jnp.float32) + ub_ref[0, :, sl]
        h = ((g * jax.nn.sigmoid(g)) * u).astype(jnp.bfloat16)
        return jnp.dot(h, dw_ref[0, sl, :].astype(jnp.bfloat16),
                       preferred_element_type=jnp.float32)

    acc = ffn_half(0) + ffn_half(1)
    prev = jnp.where(ti == 0, db_ref[0], o_ref[0])
    o_ref[0] = prev + acc


@functools.partial(jax.jit, static_argnames=("tm", "ti"))
def _ffn(x, gate_up_proj, down_proj, gate_up_proj_bias, down_proj_bias,
         tm: int, ti: int):
    e, t, h = x.shape
    i = down_proj.shape[1]
    n_ti = i // ti
    gub = gate_up_proj_bias.reshape(e, 1, 2 * i)
    db = down_proj_bias.reshape(e, 1, h)

    grid = (e, t // tm, n_ti)
    return pl.pallas_call(
        _ffn_kernel,
        grid=grid,
        in_specs=[
            pl.BlockSpec((1, tm, h), lambda ei, tmi, tii: (ei, tmi, 0)),
            pl.BlockSpec((1, h, ti), lambda ei, tmi, tii: (ei, 0, tii)),
            pl.BlockSpec((1, h, ti),
                         lambda ei, tmi, tii, n=n_ti: (ei, 0, tii + n)),
            pl.BlockSpec((1, 1, ti), lambda ei, tmi, tii: (ei, 0, tii)),
            pl.BlockSpec((1, 1, ti),
                         lambda ei, tmi, tii, n=n_ti: (ei, 0, tii + n)),
            pl.BlockSpec((1, ti, h), lambda ei, tmi, tii: (ei, tii, 0)),
            pl.BlockSpec((1, 1, h), lambda ei, tmi, tii: (ei, 0, 0)),
        ],
        out_specs=pl.BlockSpec((1, tm, h), lambda ei, tmi, tii: (ei, tmi, 0)),
        out_shape=jax.ShapeDtypeStruct((e, t, h), jnp.float32),
        compiler_params=pltpu.CompilerParams(
            dimension_semantics=("parallel", "parallel", "arbitrary")),
    )(x, gate_up_proj, gate_up_proj, gub, gub, down_proj, db)


def kernel(x, gate_up_proj, down_proj, gate_up_proj_bias, down_proj_bias):
    t = x.shape[1]
    i = down_proj.shape[1]
    tm = min(t, 1024)
    ti = min(i, 1024)
    return _ffn(x, gate_up_proj, down_proj, gate_up_proj_bias, down_proj_bias,
                tm, ti)
